# baseline (speedup 1.0000x reference)
import functools

import jax
import jax.numpy as jnp
from jax import lax
from jax.experimental import pallas as pl
from jax.experimental.pallas import tpu as pltpu
from jax.experimental.pallas import tpu_sc as plsc

B, D, C = 2048, 1024, 200
K_RANGE = (2, 3, 4, 5)
KSUM = sum(K_RANGE)
OFFS = (0, 400, 1000, 1800)
ENDS = (400, 1000, 1800, 2800)
NP_TOT = 2800
NP_PAD = 2816
BLK_B = 256
NB = B // BLK_B
NG = len(K_RANGE)

_I32_MAX = jnp.iinfo(jnp.int32).max


def _dist_kernel(x_ref, xsq_ref, p_ref, n2_ref,
                 sel_ref, cls_ref, mins_ref, counts_ref):
    pid = pl.program_id(0)
    x = x_ref[...]
    p = p_ref[...]
    s = lax.dot_general(x, p, (((1,), (1,)), ((), ())),
                        precision=lax.Precision.DEFAULT,
                        preferred_element_type=jnp.float32)
    d2 = (xsq_ref[...] + n2_ref[...]) - 2.0 * s
    li = lax.broadcasted_iota(jnp.int32, (BLK_B, NP_PAD), 1)
    ci = lax.broadcasted_iota(jnp.int32, (BLK_B, 256), 1)
    sels, clss, mins, cnts = [], [], [], []
    for g in range(NG):
        m = (li >= OFFS[g]) & (li < ENDS[g])
        dm = jnp.where(m, d2, jnp.inf)
        gmin = jnp.min(dm, axis=1, keepdims=True)
        idx = jnp.min(jnp.where(m & (dm == gmin), li, _I32_MAX),
                      axis=1, keepdims=True)
        cls = (idx - OFFS[g]) // K_RANGE[g]
        sels.append(idx)
        clss.append(cls)
        mins.append(gmin)
        cnts.append(jnp.sum((cls == ci).astype(jnp.float32),
                            axis=0, keepdims=True))
    sel_ref[...] = jnp.concatenate(sels, axis=1)
    cls_ref[...] = jnp.concatenate(clss, axis=1)
    minsum = jnp.sum(jnp.concatenate(mins, axis=1), axis=0, keepdims=True)
    mins_ref[...] = minsum.reshape(1, 1, NG)

    @pl.when(pid == 0)
    def _():
        counts_ref[...] = jnp.zeros((NG, 256), jnp.float32)

    counts_ref[...] += jnp.concatenate(cnts, axis=0)


def _dist_call(x, xsq, ppad, n2pad):
    return pl.pallas_call(
        _dist_kernel,
        grid=(NB,),
        in_specs=[
            pl.BlockSpec((BLK_B, D), lambda i: (i, 0)),
            pl.BlockSpec((BLK_B, 1), lambda i: (i, 0)),
            pl.BlockSpec((NP_PAD, D), lambda i: (0, 0)),
            pl.BlockSpec((1, NP_PAD), lambda i: (0, 0)),
        ],
        out_specs=[
            pl.BlockSpec((BLK_B, NG), lambda i: (i, 0)),
            pl.BlockSpec((BLK_B, NG), lambda i: (i, 0)),
            pl.BlockSpec((1, 1, NG), lambda i: (i, 0, 0)),
            pl.BlockSpec((NG, 256), lambda i: (0, 0)),
        ],
        out_shape=[
            jax.ShapeDtypeStruct((B, NG), jnp.int32),
            jax.ShapeDtypeStruct((B, NG), jnp.int32),
            jax.ShapeDtypeStruct((NB, 1, NG), jnp.float32),
            jax.ShapeDtypeStruct((NG, 256), jnp.float32),
        ],
    )(x, xsq, ppad, n2pad)


LCH = 704
NLCH = NP_PAD // LCH


def _loss_kernel(p_ref, cv_ref, mins_ref, out_ref, g1_acc, u_acc, s2_acc):
    pid = pl.program_id(0)
    hi = lax.Precision.HIGHEST

    @pl.when(pid == 0)
    def _():
        g1_acc[...] = jnp.zeros((D, D), jnp.float32)
        u_acc[...] = jnp.zeros((16, D), jnp.float32)
        s2_acc[...] = jnp.zeros((16, D), jnp.float32)

    p = p_ref[...]
    rr0 = lax.broadcasted_iota(jnp.int32, (LCH, 256), 0) + pid * LCH
    cc = lax.broadcasted_iota(jnp.int32, (LCH, 256), 1)
    ohs = []
    for g in range(NG):
        t = rr0 - OFFS[g] - cc * K_RANGE[g]
        ohs.append(jnp.where((t >= 0) & (t < K_RANGE[g]), 1.0, 0.0))
    oh = jnp.concatenate(ohs, axis=1)
    w = lax.dot_general(oh, cv_ref[...], (((1,), (0,)), ((), ())),
                        precision=hi, preferred_element_type=jnp.float32)
    a = p * w
    g1_acc[...] += lax.dot_general(p, a, (((0,), (0,)), ((), ())),
                                   precision=lax.Precision.DEFAULT,
                                   preferred_element_type=jnp.float32)
    jj = lax.broadcasted_iota(jnp.int32, (16, LCH), 0)
    rr = lax.broadcasted_iota(jnp.int32, (16, LCH), 1) + pid * LCH
    off = jnp.where(jj < 2, 0, jnp.where(jj < 5, 400,
                                         jnp.where(jj < 9, 1000, 1800)))
    kj = jnp.where(jj < 2, 2, jnp.where(jj < 5, 3, jnp.where(jj < 9, 4, 5)))
    kb = jnp.where(jj < 2, 0, jnp.where(jj < 5, 2, jnp.where(jj < 9, 5, 9)))
    valid = (jj < KSUM) & (rr >= off) & (rr < off + C * kj)
    slot = lax.rem(rr - off, kj) == (jj - kb)
    smat = jnp.where(valid & slot, 1.0, 0.0)
    u_acc[...] += lax.dot_general(smat, a, (((1,), (0,)), ((), ())),
                                  precision=hi,
                                  preferred_element_type=jnp.float32)
    s2_acc[...] += lax.dot_general(smat, a * p, (((1,), (0,)), ((), ())),
                                   precision=hi,
                                   preferred_element_type=jnp.float32)

    @pl.when(pid == NLCH - 1)
    def _():
        m = u_acc[...] * (1.0 / B)
        mtm = lax.dot_general(m, m, (((0,), (0,)), ((), ())),
                              precision=hi, preferred_element_type=jnp.float32)
        n_tot = B * KSUM
        cov = (g1_acc[...] - B * mtm) * (1.0 / (n_tot - 1))
        covsq = cov * cov
        ii = lax.broadcasted_iota(jnp.int32, (D, D), 0)
        ll = lax.broadcasted_iota(jnp.int32, (D, D), 1)
        cov_loss = jnp.sum(jnp.where(ii == ll, 0.0, covsq)) * (1.0 / D)
        var = (s2_acc[...] - B * (m * m)) * (1.0 / (B - 1))
        std = jnp.sqrt(var + 1e-4)
        rowok = lax.broadcasted_iota(jnp.int32, (16, D), 0) < KSUM
        std_loss = jnp.sum(jnp.where(rowok, jnp.maximum(1.0 - std, 0.0), 0.0)) \
            * (1.0 / (KSUM * D))
        repr_loss = jnp.sum(mins_ref[...]) * (1.0 / (B * NG * D))
        loss = 25.0 * repr_loss + 25.0 * std_loss + cov_loss
        out_ref[...] = jnp.reshape(loss, (1, 1))


def _loss_call(ppad, cv, mins):
    return pl.pallas_call(
        _loss_kernel,
        grid=(NLCH,),
        in_specs=[
            pl.BlockSpec((LCH, D), lambda i: (i, 0)),
            pl.BlockSpec((1024, 1), lambda i: (0, 0)),
            pl.BlockSpec((NB, 1, NG), lambda i: (0, 0, 0)),
        ],
        out_specs=pl.BlockSpec((1, 1), lambda i: (0, 0)),
        out_shape=jax.ShapeDtypeStruct((1, 1), jnp.float32),
        scratch_shapes=[
            pltpu.VMEM((D, D), jnp.float32),
            pltpu.VMEM((16, D), jnp.float32),
            pltpu.VMEM((16, D), jnp.float32),
        ],
    )(ppad, cv, mins)


NBW = B // 32


def _gather_body(p_hbm, idxn_hbm, idxp_hbm, outn_hbm, outp_hbm,
                 idx_v, bA, bB, gsemA, gsemB, osemA, osemB):
    wid = lax.axis_index("s") * 2 + lax.axis_index("c")
    b0 = pl.multiple_of(wid * NBW, NBW)
    NOFF = KSUM * NBW

    for j in range(KSUM):
        pltpu.async_copy(
            idxp_hbm.at[pl.ds(pl.multiple_of(j * B + wid * NBW, 64), NBW)],
            idx_v.at[pl.ds(j * NBW, NBW)], gsemA)
    pltpu.async_copy(
        idxn_hbm.at[pl.ds(pl.multiple_of(wid * NBW * 8, 8), NBW * 8)],
        idx_v.at[pl.ds(NOFF, NBW * 8)], gsemA)
    for j in range(KSUM):
        pltpu.make_async_copy(idxp_hbm.at[pl.ds(0, NBW)],
                              idx_v.at[pl.ds(0, NBW)], gsemA).wait()
    pltpu.make_async_copy(idxn_hbm.at[pl.ds(0, NBW * 8)],
                          idx_v.at[pl.ds(0, NBW * 8)], gsemA).wait()

    pA32 = bA
    pB32 = bB

    def p_gather(c, buf, sem):
        j = c // 2
        half = c - 2 * j
        st = pl.multiple_of(j * NBW + 32 * half, 32)
        pltpu.async_copy(p_hbm.at[idx_v.at[pl.ds(st, 32)]], buf, sem)

    def p_gwait(buf, sem):
        pltpu.make_async_copy(p_hbm.at[idx_v.at[pl.ds(0, 32)]],
                              buf, sem).wait()

    def p_write(c, buf, sem):
        j = c // 2
        half = c - 2 * j
        pltpu.async_copy(buf, outp_hbm.at[j, pl.ds(b0 + 32 * half, 32)],
                         sem)

    def p_wdrain(sem):
        pltpu.make_async_copy(pA32, outp_hbm.at[0, pl.ds(b0, 32)],
                              sem).wait()

    p_gather(0, pA32, gsemA)

    def bodyp(t, carry):
        c0 = 2 * t

        @pl.when(t > 0)
        def _():
            p_wdrain(osemB)

        p_gather(c0 + 1, pB32, gsemB)
        p_gwait(pA32, gsemA)
        p_write(c0, pA32, osemA)

        @pl.when(t < KSUM - 1)
        def _():
            p_wdrain(osemA)
            p_gather(c0 + 2, pA32, gsemA)

        p_gwait(pB32, gsemB)
        p_write(c0 + 1, pB32, osemB)
        return carry

    lax.fori_loop(0, KSUM, bodyp, 0)
    p_wdrain(osemA)
    p_wdrain(osemB)

    def n_gather(c, buf, sem):
        st = pl.multiple_of(NOFF + c * 32, 32)
        pltpu.async_copy(p_hbm.at[idx_v.at[pl.ds(st, 32)]], buf, sem)

    def n_gwait(buf, sem):
        pltpu.make_async_copy(p_hbm.at[idx_v.at[pl.ds(NOFF, 32)]],
                              buf, sem).wait()

    def n_write(c, buf, sem):
        bs = b0 + c * 4
        for j in range(4):
            pltpu.async_copy(buf.at[pl.ds(j * 8, NG)],
                             outn_hbm.at[bs + j], sem)

    def n_wdrain(sem):
        for j in range(4):
            pltpu.make_async_copy(bA.at[pl.ds(0, NG)],
                                  outn_hbm.at[b0], sem).wait()

    n_gather(0, bA, gsemA)

    def bodyn(t, carry):
        c0 = 2 * t

        @pl.when(t > 0)
        def _():
            n_wdrain(osemB)

        n_gather(c0 + 1, bB, gsemB)
        n_gwait(bA, gsemA)
        n_write(c0, bA, osemA)

        @pl.when(t < NBW // 8 - 1)
        def _():
            n_wdrain(osemA)
            n_gather(c0 + 2, bA, gsemA)

        n_gwait(bB, gsemB)
        n_write(c0 + 1, bB, osemB)
        return carry

    lax.fori_loop(0, NBW // 8, bodyn, 0)
    n_wdrain(osemA)
    n_wdrain(osemB)


def _gather_call(ppad, idx_n8, idx_pjT):
    mesh = plsc.VectorSubcoreMesh(core_axis_name="c", subcore_axis_name="s")
    f = pl.kernel(
        _gather_body,
        out_type=[
            jax.ShapeDtypeStruct((B, NG, D), jnp.float32),
            jax.ShapeDtypeStruct((KSUM, B, D), jnp.float32),
        ],
        mesh=mesh,
        scratch_types=[
            pltpu.VMEM((NBW * (KSUM + 8),), jnp.int32),
            pltpu.VMEM((32, D), jnp.float32),
            pltpu.VMEM((32, D), jnp.float32),
            pltpu.SemaphoreType.DMA,
            pltpu.SemaphoreType.DMA,
            pltpu.SemaphoreType.DMA,
            pltpu.SemaphoreType.DMA,
        ],
    )
    return f(ppad, idx_n8, idx_pjT)


def kernel(x, P2, P3, P4, P5):
    groups = (P2, P3, P4, P5)
    flats = [g.reshape(C * k, D) for g, k in zip(groups, K_RANGE)]
    n2 = jnp.concatenate([jnp.sum(f ** 2, axis=1) for f in flats])
    ppad = jnp.concatenate(
        flats + [jnp.zeros((NP_PAD - NP_TOT, D), jnp.float32)], axis=0)
    n2pad = jnp.pad(n2, (0, NP_PAD - NP_TOT)).reshape(1, NP_PAD)
    xsq = jnp.sum(x ** 2, axis=1, keepdims=True)

    sel, cls, mins, counts = _dist_call(x, xsq, ppad, n2pad)

    loss = _loss_call(ppad, counts.reshape(1024, 1), mins)[0, 0]

    idx_n8 = jnp.pad(sel, ((0, 0), (0, 8 - NG))).reshape(-1)
    idx_p = jnp.concatenate(
        [OFFS[g] + cls[:, g:g + 1] * K_RANGE[g]
         + jnp.arange(K_RANGE[g], dtype=jnp.int32)[None, :]
         for g in range(NG)], axis=1)
    idx_pjT = idx_p.T.reshape(-1)

    nearest_prototypes, proto_jb = _gather_call(ppad, idx_n8, idx_pjT)
    prototype_set = jnp.transpose(proto_jb, (1, 0, 2))
    class_indices = cls[:, NG - 1]
    return loss, nearest_prototypes, prototype_set, class_indices


# speedup vs baseline: 2.2071x; 2.2071x over previous
import functools

import jax
import jax.numpy as jnp
from jax import lax
from jax.experimental import pallas as pl
from jax.experimental.pallas import tpu as pltpu
from jax.experimental.pallas import tpu_sc as plsc

B, D, C = 2048, 1024, 200
K_RANGE = (2, 3, 4, 5)
KSUM = sum(K_RANGE)
OFFS = (0, 400, 1000, 1800)
ENDS = (400, 1000, 1800, 2800)
NP_TOT = 2800
NP_PAD = 2816
BLK_B = 256
NB = B // BLK_B
NG = len(K_RANGE)

_I32_MAX = jnp.iinfo(jnp.int32).max


def _dist_kernel(x_ref, xsq_ref, p_ref, n2_ref,
                 sel_ref, cls_ref, mins_ref, counts_ref):
    pid = pl.program_id(0)
    x = x_ref[...]
    p = p_ref[...]
    s = lax.dot_general(x, p, (((1,), (1,)), ((), ())),
                        precision=lax.Precision.DEFAULT,
                        preferred_element_type=jnp.float32)
    d2 = (xsq_ref[...] + n2_ref[...]) - 2.0 * s
    li = lax.broadcasted_iota(jnp.int32, (BLK_B, NP_PAD), 1)
    ci = lax.broadcasted_iota(jnp.int32, (BLK_B, 256), 1)
    sels, clss, mins, cnts = [], [], [], []
    for g in range(NG):
        m = (li >= OFFS[g]) & (li < ENDS[g])
        dm = jnp.where(m, d2, jnp.inf)
        gmin = jnp.min(dm, axis=1, keepdims=True)
        idx = jnp.min(jnp.where(m & (dm == gmin), li, _I32_MAX),
                      axis=1, keepdims=True)
        cls = (idx - OFFS[g]) // K_RANGE[g]
        sels.append(idx)
        clss.append(cls)
        mins.append(gmin)
        cnts.append(jnp.sum((cls == ci).astype(jnp.float32),
                            axis=0, keepdims=True))
    sel_ref[...] = jnp.concatenate(sels, axis=1)
    cls_ref[...] = jnp.concatenate(clss, axis=1)
    minsum = jnp.sum(jnp.concatenate(mins, axis=1), axis=0, keepdims=True)
    mins_ref[...] = minsum.reshape(1, 1, NG)

    @pl.when(pid == 0)
    def _():
        counts_ref[...] = jnp.zeros((NG, 256), jnp.float32)

    counts_ref[...] += jnp.concatenate(cnts, axis=0)


def _dist_call(x, xsq, ppad, n2pad):
    return pl.pallas_call(
        _dist_kernel,
        grid=(NB,),
        in_specs=[
            pl.BlockSpec((BLK_B, D), lambda i: (i, 0)),
            pl.BlockSpec((BLK_B, 1), lambda i: (i, 0)),
            pl.BlockSpec((NP_PAD, D), lambda i: (0, 0)),
            pl.BlockSpec((1, NP_PAD), lambda i: (0, 0)),
        ],
        out_specs=[
            pl.BlockSpec((BLK_B, NG), lambda i: (i, 0)),
            pl.BlockSpec((BLK_B, NG), lambda i: (i, 0)),
            pl.BlockSpec((1, 1, NG), lambda i: (i, 0, 0)),
            pl.BlockSpec((NG, 256), lambda i: (0, 0)),
        ],
        out_shape=[
            jax.ShapeDtypeStruct((B, NG), jnp.int32),
            jax.ShapeDtypeStruct((B, NG), jnp.int32),
            jax.ShapeDtypeStruct((NB, 1, NG), jnp.float32),
            jax.ShapeDtypeStruct((NG, 256), jnp.float32),
        ],
    )(x, xsq, ppad, n2pad)


LCH = 704
NLCH = NP_PAD // LCH


def _loss_kernel(p_ref, cv_ref, mins_ref, out_ref, g1_acc, u_acc, s2_acc):
    pid = pl.program_id(0)
    hi = lax.Precision.HIGHEST

    @pl.when(pid == 0)
    def _():
        g1_acc[...] = jnp.zeros((D, D), jnp.float32)
        u_acc[...] = jnp.zeros((16, D), jnp.float32)
        s2_acc[...] = jnp.zeros((16, D), jnp.float32)

    p = p_ref[...]
    rr0 = lax.broadcasted_iota(jnp.int32, (LCH, 256), 0) + pid * LCH
    cc = lax.broadcasted_iota(jnp.int32, (LCH, 256), 1)
    ohs = []
    for g in range(NG):
        t = rr0 - OFFS[g] - cc * K_RANGE[g]
        ohs.append(jnp.where((t >= 0) & (t < K_RANGE[g]), 1.0, 0.0))
    oh = jnp.concatenate(ohs, axis=1)
    w = lax.dot_general(oh, cv_ref[...], (((1,), (0,)), ((), ())),
                        precision=hi, preferred_element_type=jnp.float32)
    a = p * w
    g1_acc[...] += lax.dot_general(p, a, (((0,), (0,)), ((), ())),
                                   precision=lax.Precision.DEFAULT,
                                   preferred_element_type=jnp.float32)
    jj = lax.broadcasted_iota(jnp.int32, (16, LCH), 0)
    rr = lax.broadcasted_iota(jnp.int32, (16, LCH), 1) + pid * LCH
    off = jnp.where(jj < 2, 0, jnp.where(jj < 5, 400,
                                         jnp.where(jj < 9, 1000, 1800)))
    kj = jnp.where(jj < 2, 2, jnp.where(jj < 5, 3, jnp.where(jj < 9, 4, 5)))
    kb = jnp.where(jj < 2, 0, jnp.where(jj < 5, 2, jnp.where(jj < 9, 5, 9)))
    valid = (jj < KSUM) & (rr >= off) & (rr < off + C * kj)
    slot = lax.rem(rr - off, kj) == (jj - kb)
    smat = jnp.where(valid & slot, 1.0, 0.0)
    u_acc[...] += lax.dot_general(smat, a, (((1,), (0,)), ((), ())),
                                  precision=hi,
                                  preferred_element_type=jnp.float32)
    s2_acc[...] += lax.dot_general(smat, a * p, (((1,), (0,)), ((), ())),
                                   precision=hi,
                                   preferred_element_type=jnp.float32)

    @pl.when(pid == NLCH - 1)
    def _():
        m = u_acc[...] * (1.0 / B)
        mtm = lax.dot_general(m, m, (((0,), (0,)), ((), ())),
                              precision=hi, preferred_element_type=jnp.float32)
        n_tot = B * KSUM
        cov = (g1_acc[...] - B * mtm) * (1.0 / (n_tot - 1))
        covsq = cov * cov
        ii = lax.broadcasted_iota(jnp.int32, (D, D), 0)
        ll = lax.broadcasted_iota(jnp.int32, (D, D), 1)
        cov_loss = jnp.sum(jnp.where(ii == ll, 0.0, covsq)) * (1.0 / D)
        var = (s2_acc[...] - B * (m * m)) * (1.0 / (B - 1))
        std = jnp.sqrt(var + 1e-4)
        rowok = lax.broadcasted_iota(jnp.int32, (16, D), 0) < KSUM
        std_loss = jnp.sum(jnp.where(rowok, jnp.maximum(1.0 - std, 0.0), 0.0)) \
            * (1.0 / (KSUM * D))
        repr_loss = jnp.sum(mins_ref[...]) * (1.0 / (B * NG * D))
        loss = 25.0 * repr_loss + 25.0 * std_loss + cov_loss
        out_ref[...] = jnp.reshape(loss, (1, 1))


def _loss_call(ppad, cv, mins):
    return pl.pallas_call(
        _loss_kernel,
        grid=(NLCH,),
        in_specs=[
            pl.BlockSpec((LCH, D), lambda i: (i, 0)),
            pl.BlockSpec((1024, 1), lambda i: (0, 0)),
            pl.BlockSpec((NB, 1, NG), lambda i: (0, 0, 0)),
        ],
        out_specs=pl.BlockSpec((1, 1), lambda i: (0, 0)),
        out_shape=jax.ShapeDtypeStruct((1, 1), jnp.float32),
        scratch_shapes=[
            pltpu.VMEM((D, D), jnp.float32),
            pltpu.VMEM((16, D), jnp.float32),
            pltpu.VMEM((16, D), jnp.float32),
        ],
    )(ppad, cv, mins)


NBW = B // 32


def _gather_body(p_hbm, idxn_hbm, idxp_hbm, outn_hbm, outp_hbm,
                 idx_v, n0, n1, n2, n3, bA, bB, gsemA, gsemB, osemA, osemB):
    wid = lax.axis_index("s") * 2 + lax.axis_index("c")
    b0 = pl.multiple_of(wid * NBW, NBW)
    NOFF = KSUM * NBW

    for j in range(KSUM):
        pltpu.async_copy(
            idxp_hbm.at[pl.ds(pl.multiple_of(j * B + wid * NBW, 64), NBW)],
            idx_v.at[pl.ds(j * NBW, NBW)], gsemA)
    pltpu.async_copy(
        idxn_hbm.at[pl.ds(pl.multiple_of(wid * NBW * 8, 8), NBW * 8)],
        idx_v.at[pl.ds(NOFF, NBW * 8)], gsemA)
    for j in range(KSUM):
        pltpu.make_async_copy(idxp_hbm.at[pl.ds(0, NBW)],
                              idx_v.at[pl.ds(0, NBW)], gsemA).wait()
    pltpu.make_async_copy(idxn_hbm.at[pl.ds(0, NBW * 8)],
                          idx_v.at[pl.ds(0, NBW * 8)], gsemA).wait()

    pA32 = bA
    pB32 = bB

    def p_gather(c, buf, sem):
        j = c // 2
        half = c - 2 * j
        st = pl.multiple_of(j * NBW + 32 * half, 32)
        pltpu.async_copy(p_hbm.at[idx_v.at[pl.ds(st, 32)]], buf, sem)

    def p_gwait(buf, sem):
        pltpu.make_async_copy(p_hbm.at[idx_v.at[pl.ds(0, 32)]],
                              buf, sem).wait()

    def p_write(c, buf, sem):
        j = c // 2
        half = c - 2 * j
        pltpu.async_copy(buf, outp_hbm.at[j, pl.ds(b0 + 32 * half, 32)],
                         sem)

    def p_wdrain(sem):
        pltpu.make_async_copy(pA32, outp_hbm.at[0, pl.ds(b0, 32)],
                              sem).wait()

    p_gather(0, pA32, gsemA)

    def bodyp(t, carry):
        c0 = 2 * t

        @pl.when(t > 0)
        def _():
            p_wdrain(osemB)

        p_gather(c0 + 1, pB32, gsemB)
        p_gwait(pA32, gsemA)
        p_write(c0, pA32, osemA)

        @pl.when(t < KSUM - 1)
        def _():
            p_wdrain(osemA)
            p_gather(c0 + 2, pA32, gsemA)

        p_gwait(pB32, gsemB)
        p_write(c0 + 1, pB32, osemB)
        return carry

    lax.fori_loop(0, KSUM, bodyp, 0)
    p_wdrain(osemA)
    p_wdrain(osemB)

    def n_gather(t, bx, by, sem):
        st = pl.multiple_of(NOFF + 16 * t, 8)
        pltpu.async_copy(p_hbm.at[idx_v.at[pl.ds(st, NG)]], bx, sem)
        pltpu.async_copy(p_hbm.at[idx_v.at[pl.ds(st + 8, NG)]], by, sem)

    def n_gwait(bx, by, sem):
        pltpu.make_async_copy(p_hbm.at[idx_v.at[pl.ds(NOFF, NG)]],
                              bx, sem).wait()
        pltpu.make_async_copy(p_hbm.at[idx_v.at[pl.ds(NOFF, NG)]],
                              by, sem).wait()

    def n_wdrain(sem):
        pltpu.make_async_copy(n0, outn_hbm.at[b0], sem).wait()
        pltpu.make_async_copy(n1, outn_hbm.at[b0], sem).wait()

    n_gather(0, n0, n1, gsemA)

    def bodyn(t, carry):
        b = b0 + 2 * t
        even = lax.rem(t, 2) == 0
        more = t < NBW // 2 - 1

        @pl.when(even)
        def _():
            @pl.when(more)
            def _():
                @pl.when(t > 0)
                def _():
                    n_wdrain(osemB)

                n_gather(t + 1, n2, n3, gsemB)

            n_gwait(n0, n1, gsemA)
            pltpu.async_copy(n0, outn_hbm.at[b], osemA)
            pltpu.async_copy(n1, outn_hbm.at[b + 1], osemA)

        @pl.when(jnp.logical_not(even))
        def _():
            @pl.when(more)
            def _():
                n_wdrain(osemA)
                n_gather(t + 1, n0, n1, gsemA)

            n_gwait(n2, n3, gsemB)
            pltpu.async_copy(n2, outn_hbm.at[b], osemB)
            pltpu.async_copy(n3, outn_hbm.at[b + 1], osemB)

        return carry

    lax.fori_loop(0, NBW // 2, bodyn, 0)
    n_wdrain(osemA)
    n_wdrain(osemB)


def _gather_call(ppad, idx_n8, idx_pjT):
    mesh = plsc.VectorSubcoreMesh(core_axis_name="c", subcore_axis_name="s")
    f = pl.kernel(
        _gather_body,
        out_type=[
            jax.ShapeDtypeStruct((B, NG, D), jnp.float32),
            jax.ShapeDtypeStruct((KSUM, B, D), jnp.float32),
        ],
        mesh=mesh,
        scratch_types=[
            pltpu.VMEM((NBW * (KSUM + 8),), jnp.int32),
            pltpu.VMEM((NG, D), jnp.float32),
            pltpu.VMEM((NG, D), jnp.float32),
            pltpu.VMEM((NG, D), jnp.float32),
            pltpu.VMEM((NG, D), jnp.float32),
            pltpu.VMEM((32, D), jnp.float32),
            pltpu.VMEM((32, D), jnp.float32),
            pltpu.SemaphoreType.DMA,
            pltpu.SemaphoreType.DMA,
            pltpu.SemaphoreType.DMA,
            pltpu.SemaphoreType.DMA,
        ],
    )
    return f(ppad, idx_n8, idx_pjT)


def kernel(x, P2, P3, P4, P5):
    groups = (P2, P3, P4, P5)
    flats = [g.reshape(C * k, D) for g, k in zip(groups, K_RANGE)]
    n2 = jnp.concatenate([jnp.sum(f ** 2, axis=1) for f in flats])
    ppad = jnp.concatenate(
        flats + [jnp.zeros((NP_PAD - NP_TOT, D), jnp.float32)], axis=0)
    n2pad = jnp.pad(n2, (0, NP_PAD - NP_TOT)).reshape(1, NP_PAD)
    xsq = jnp.sum(x ** 2, axis=1, keepdims=True)

    sel, cls, mins, counts = _dist_call(x, xsq, ppad, n2pad)

    loss = _loss_call(ppad, counts.reshape(1024, 1), mins)[0, 0]

    idx_n8 = jnp.pad(sel, ((0, 0), (0, 8 - NG))).reshape(-1)
    idx_p = jnp.concatenate(
        [OFFS[g] + cls[:, g:g + 1] * K_RANGE[g]
         + jnp.arange(K_RANGE[g], dtype=jnp.int32)[None, :]
         for g in range(NG)], axis=1)
    idx_pjT = idx_p.T.reshape(-1)

    nearest_prototypes, proto_jb = _gather_call(ppad, idx_n8, idx_pjT)
    prototype_set = jnp.transpose(proto_jb, (1, 0, 2))
    class_indices = cls[:, NG - 1]
    return loss, nearest_prototypes, prototype_set, class_indices


# speedup vs baseline: 2.2232x; 1.0073x over previous
import functools

import jax
import jax.numpy as jnp
from jax import lax
from jax.experimental import pallas as pl
from jax.experimental.pallas import tpu as pltpu
from jax.experimental.pallas import tpu_sc as plsc

B, D, C = 2048, 1024, 200
K_RANGE = (2, 3, 4, 5)
KSUM = sum(K_RANGE)
OFFS = (0, 400, 1000, 1800)
ENDS = (400, 1000, 1800, 2800)
NP_TOT = 2800
NP_PAD = 2816
BLK_B = 512
NB = B // BLK_B
NG = len(K_RANGE)

_I32_MAX = jnp.iinfo(jnp.int32).max


def _dist_kernel(x_ref, xsq_ref, p_ref, n2_ref,
                 sel_ref, cls_ref, mins_ref, counts_ref):
    pid = pl.program_id(0)
    x = x_ref[...]
    p = p_ref[...]
    s = lax.dot_general(x, p, (((1,), (1,)), ((), ())),
                        precision=lax.Precision.DEFAULT,
                        preferred_element_type=jnp.float32)
    d2 = (xsq_ref[...] + n2_ref[...]) - 2.0 * s
    li = lax.broadcasted_iota(jnp.int32, (BLK_B, NP_PAD), 1)
    ci = lax.broadcasted_iota(jnp.int32, (BLK_B, 256), 1)
    sels, clss, mins, cnts = [], [], [], []
    for g in range(NG):
        m = (li >= OFFS[g]) & (li < ENDS[g])
        dm = jnp.where(m, d2, jnp.inf)
        gmin = jnp.min(dm, axis=1, keepdims=True)
        idx = jnp.min(jnp.where(m & (dm == gmin), li, _I32_MAX),
                      axis=1, keepdims=True)
        cls = (idx - OFFS[g]) // K_RANGE[g]
        sels.append(idx)
        clss.append(cls)
        mins.append(gmin)
        cnts.append(jnp.sum((cls == ci).astype(jnp.float32),
                            axis=0, keepdims=True))
    sel_ref[...] = jnp.concatenate(sels + sels, axis=1)
    cls_ref[...] = jnp.concatenate(clss, axis=1)
    minsum = jnp.sum(jnp.concatenate(mins, axis=1), axis=0, keepdims=True)
    mins_ref[...] = minsum.reshape(1, 1, NG)

    @pl.when(pid == 0)
    def _():
        counts_ref[...] = jnp.zeros((NG, 256), jnp.float32)

    counts_ref[...] += jnp.concatenate(cnts, axis=0)


def _dist_call(x, xsq, ppad, n2pad):
    return pl.pallas_call(
        _dist_kernel,
        grid=(NB,),
        in_specs=[
            pl.BlockSpec((BLK_B, D), lambda i: (i, 0)),
            pl.BlockSpec((BLK_B, 1), lambda i: (i, 0)),
            pl.BlockSpec((NP_PAD, D), lambda i: (0, 0)),
            pl.BlockSpec((1, NP_PAD), lambda i: (0, 0)),
        ],
        out_specs=[
            pl.BlockSpec((BLK_B, 2 * NG), lambda i: (i, 0)),
            pl.BlockSpec((BLK_B, NG), lambda i: (i, 0)),
            pl.BlockSpec((1, 1, NG), lambda i: (i, 0, 0)),
            pl.BlockSpec((NG, 256), lambda i: (0, 0)),
        ],
        out_shape=[
            jax.ShapeDtypeStruct((B, 2 * NG), jnp.int32),
            jax.ShapeDtypeStruct((B, NG), jnp.int32),
            jax.ShapeDtypeStruct((NB, 1, NG), jnp.float32),
            jax.ShapeDtypeStruct((NG, 256), jnp.float32),
        ],
    )(x, xsq, ppad, n2pad)


LCH = 704
NLCH = NP_PAD // LCH


def _loss_kernel(p_ref, cv_ref, mins_ref, out_ref, g1_acc, u_acc, s2_acc):
    pid = pl.program_id(0)
    hi = lax.Precision.HIGHEST

    @pl.when(pid == 0)
    def _():
        g1_acc[...] = jnp.zeros((D, D), jnp.float32)
        u_acc[...] = jnp.zeros((16, D), jnp.float32)
        s2_acc[...] = jnp.zeros((16, D), jnp.float32)

    p = p_ref[...]
    rr0 = lax.broadcasted_iota(jnp.int32, (LCH, 256), 0) + pid * LCH
    cc = lax.broadcasted_iota(jnp.int32, (LCH, 256), 1)
    ohs = []
    for g in range(NG):
        t = rr0 - OFFS[g] - cc * K_RANGE[g]
        ohs.append(jnp.where((t >= 0) & (t < K_RANGE[g]), 1.0, 0.0))
    oh = jnp.concatenate(ohs, axis=1)
    w = lax.dot_general(oh, cv_ref[...], (((1,), (0,)), ((), ())),
                        precision=hi, preferred_element_type=jnp.float32)
    a = p * w
    g1_acc[...] += lax.dot_general(p, a, (((0,), (0,)), ((), ())),
                                   precision=lax.Precision.DEFAULT,
                                   preferred_element_type=jnp.float32)
    jj = lax.broadcasted_iota(jnp.int32, (16, LCH), 0)
    rr = lax.broadcasted_iota(jnp.int32, (16, LCH), 1) + pid * LCH
    off = jnp.where(jj < 2, 0, jnp.where(jj < 5, 400,
                                         jnp.where(jj < 9, 1000, 1800)))
    kj = jnp.where(jj < 2, 2, jnp.where(jj < 5, 3, jnp.where(jj < 9, 4, 5)))
    kb = jnp.where(jj < 2, 0, jnp.where(jj < 5, 2, jnp.where(jj < 9, 5, 9)))
    valid = (jj < KSUM) & (rr >= off) & (rr < off + C * kj)
    slot = lax.rem(rr - off, kj) == (jj - kb)
    smat = jnp.where(valid & slot, 1.0, 0.0)
    u_acc[...] += lax.dot_general(smat, a, (((1,), (0,)), ((), ())),
                                  precision=hi,
                                  preferred_element_type=jnp.float32)
    s2_acc[...] += lax.dot_general(smat, a * p, (((1,), (0,)), ((), ())),
                                   precision=hi,
                                   preferred_element_type=jnp.float32)

    @pl.when(pid == NLCH - 1)
    def _():
        m = u_acc[...] * (1.0 / B)
        mtm = lax.dot_general(m, m, (((0,), (0,)), ((), ())),
                              precision=hi, preferred_element_type=jnp.float32)
        n_tot = B * KSUM
        cov = (g1_acc[...] - B * mtm) * (1.0 / (n_tot - 1))
        covsq = cov * cov
        ii = lax.broadcasted_iota(jnp.int32, (D, D), 0)
        ll = lax.broadcasted_iota(jnp.int32, (D, D), 1)
        cov_loss = jnp.sum(jnp.where(ii == ll, 0.0, covsq)) * (1.0 / D)
        var = (s2_acc[...] - B * (m * m)) * (1.0 / (B - 1))
        std = jnp.sqrt(var + 1e-4)
        rowok = lax.broadcasted_iota(jnp.int32, (16, D), 0) < KSUM
        std_loss = jnp.sum(jnp.where(rowok, jnp.maximum(1.0 - std, 0.0), 0.0)) \
            * (1.0 / (KSUM * D))
        repr_loss = jnp.sum(mins_ref[...]) * (1.0 / (B * NG * D))
        loss = 25.0 * repr_loss + 25.0 * std_loss + cov_loss
        out_ref[...] = jnp.reshape(loss, (1, 1))


def _loss_call(ppad, cv, mins):
    return pl.pallas_call(
        _loss_kernel,
        grid=(NLCH,),
        in_specs=[
            pl.BlockSpec((LCH, D), lambda i: (i, 0)),
            pl.BlockSpec((1024, 1), lambda i: (0, 0)),
            pl.BlockSpec((NB, 1, NG), lambda i: (0, 0, 0)),
        ],
        out_specs=pl.BlockSpec((1, 1), lambda i: (0, 0)),
        out_shape=jax.ShapeDtypeStruct((1, 1), jnp.float32),
        scratch_shapes=[
            pltpu.VMEM((D, D), jnp.float32),
            pltpu.VMEM((16, D), jnp.float32),
            pltpu.VMEM((16, D), jnp.float32),
        ],
    )(ppad, cv, mins)


NBW = B // 32


def _gather_body(p_hbm, idxn_hbm, idxp_hbm, outn_hbm, outp_hbm,
                 idx_v, n0, n1, n2, n3, bA, bB, gsemA, gsemB, osemA, osemB):
    wid = lax.axis_index("s") * 2 + lax.axis_index("c")
    b0 = pl.multiple_of(wid * NBW, NBW)
    NOFF = KSUM * NBW

    for j in range(KSUM):
        pltpu.async_copy(
            idxp_hbm.at[pl.ds(pl.multiple_of(j * B + wid * NBW, 64), NBW)],
            idx_v.at[pl.ds(j * NBW, NBW)], gsemA)
    pltpu.async_copy(
        idxn_hbm.at[pl.ds(pl.multiple_of(wid * NBW * 8, 8), NBW * 8)],
        idx_v.at[pl.ds(NOFF, NBW * 8)], gsemA)
    for j in range(KSUM):
        pltpu.make_async_copy(idxp_hbm.at[pl.ds(0, NBW)],
                              idx_v.at[pl.ds(0, NBW)], gsemA).wait()
    pltpu.make_async_copy(idxn_hbm.at[pl.ds(0, NBW * 8)],
                          idx_v.at[pl.ds(0, NBW * 8)], gsemA).wait()

    pA32 = bA
    pB32 = bB

    def p_gather(c, buf, sem):
        j = c // 2
        half = c - 2 * j
        st = pl.multiple_of(j * NBW + 32 * half, 32)
        pltpu.async_copy(p_hbm.at[idx_v.at[pl.ds(st, 32)]], buf, sem)

    def p_gwait(buf, sem):
        pltpu.make_async_copy(p_hbm.at[idx_v.at[pl.ds(0, 32)]],
                              buf, sem).wait()

    def p_write(c, buf, sem):
        j = c // 2
        half = c - 2 * j
        pltpu.async_copy(buf, outp_hbm.at[j, pl.ds(b0 + 32 * half, 32)],
                         sem)

    def p_wdrain(sem):
        pltpu.make_async_copy(pA32, outp_hbm.at[0, pl.ds(b0, 32)],
                              sem).wait()

    p_gather(0, pA32, gsemA)

    def bodyp(t, carry):
        c0 = 2 * t

        @pl.when(t > 0)
        def _():
            p_wdrain(osemB)

        p_gather(c0 + 1, pB32, gsemB)
        p_gwait(pA32, gsemA)
        p_write(c0, pA32, osemA)

        @pl.when(t < KSUM - 1)
        def _():
            p_wdrain(osemA)
            p_gather(c0 + 2, pA32, gsemA)

        p_gwait(pB32, gsemB)
        p_write(c0 + 1, pB32, osemB)
        return carry

    lax.fori_loop(0, KSUM, bodyp, 0)
    p_wdrain(osemA)
    p_wdrain(osemB)

    def n_gather(t, bx, by, sem):
        st = pl.multiple_of(NOFF + 16 * t, 8)
        pltpu.async_copy(p_hbm.at[idx_v.at[pl.ds(st, NG)]], bx, sem)
        pltpu.async_copy(p_hbm.at[idx_v.at[pl.ds(st + 8, NG)]], by, sem)

    def n_gwait(bx, by, sem):
        pltpu.make_async_copy(p_hbm.at[idx_v.at[pl.ds(NOFF, NG)]],
                              bx, sem).wait()
        pltpu.make_async_copy(p_hbm.at[idx_v.at[pl.ds(NOFF, NG)]],
                              by, sem).wait()

    def n_wdrain(sem):
        pltpu.make_async_copy(n0, outn_hbm.at[b0], sem).wait()
        pltpu.make_async_copy(n1, outn_hbm.at[b0], sem).wait()

    n_gather(0, n0, n1, gsemA)

    def bodyn(t, carry):
        b = b0 + 2 * t
        even = lax.rem(t, 2) == 0
        more = t < NBW // 2 - 1

        @pl.when(even)
        def _():
            @pl.when(more)
            def _():
                @pl.when(t > 0)
                def _():
                    n_wdrain(osemB)

                n_gather(t + 1, n2, n3, gsemB)

            n_gwait(n0, n1, gsemA)
            pltpu.async_copy(n0, outn_hbm.at[b], osemA)
            pltpu.async_copy(n1, outn_hbm.at[b + 1], osemA)

        @pl.when(jnp.logical_not(even))
        def _():
            @pl.when(more)
            def _():
                n_wdrain(osemA)
                n_gather(t + 1, n0, n1, gsemA)

            n_gwait(n2, n3, gsemB)
            pltpu.async_copy(n2, outn_hbm.at[b], osemB)
            pltpu.async_copy(n3, outn_hbm.at[b + 1], osemB)

        return carry

    lax.fori_loop(0, NBW // 2, bodyn, 0)
    n_wdrain(osemA)
    n_wdrain(osemB)


def _gather_call(ppad, idx_n8, idx_pjT):
    mesh = plsc.VectorSubcoreMesh(core_axis_name="c", subcore_axis_name="s")
    f = pl.kernel(
        _gather_body,
        out_type=[
            jax.ShapeDtypeStruct((B, NG, D), jnp.float32),
            jax.ShapeDtypeStruct((KSUM, B, D), jnp.float32),
        ],
        mesh=mesh,
        scratch_types=[
            pltpu.VMEM((NBW * (KSUM + 8),), jnp.int32),
            pltpu.VMEM((NG, D), jnp.float32),
            pltpu.VMEM((NG, D), jnp.float32),
            pltpu.VMEM((NG, D), jnp.float32),
            pltpu.VMEM((NG, D), jnp.float32),
            pltpu.VMEM((32, D), jnp.float32),
            pltpu.VMEM((32, D), jnp.float32),
            pltpu.SemaphoreType.DMA,
            pltpu.SemaphoreType.DMA,
            pltpu.SemaphoreType.DMA,
            pltpu.SemaphoreType.DMA,
        ],
    )
    return f(ppad, idx_n8, idx_pjT)


def kernel(x, P2, P3, P4, P5):
    groups = (P2, P3, P4, P5)
    flats = [g.reshape(C * k, D) for g, k in zip(groups, K_RANGE)]
    n2 = jnp.concatenate([jnp.sum(f ** 2, axis=1) for f in flats])
    ppad = jnp.concatenate(
        flats + [jnp.zeros((NP_PAD - NP_TOT, D), jnp.float32)], axis=0)
    n2pad = jnp.pad(n2, (0, NP_PAD - NP_TOT)).reshape(1, NP_PAD)
    xsq = jnp.sum(x ** 2, axis=1, keepdims=True)

    sel, cls, mins, counts = _dist_call(x, xsq, ppad, n2pad)

    loss = _loss_call(ppad, counts.reshape(1024, 1), mins)[0, 0]

    idx_n8 = sel.reshape(-1)
    idx_p = jnp.concatenate(
        [OFFS[g] + cls[:, g:g + 1] * K_RANGE[g]
         + jnp.arange(K_RANGE[g], dtype=jnp.int32)[None, :]
         for g in range(NG)], axis=1)
    idx_pjT = idx_p.T.reshape(-1)

    nearest_prototypes, proto_jb = _gather_call(ppad, idx_n8, idx_pjT)
    prototype_set = jnp.transpose(proto_jb, (1, 0, 2))
    class_indices = cls[:, NG - 1]
    return loss, nearest_prototypes, prototype_set, class_indices
